# speedup vs baseline: 61.0376x; 61.0376x over previous
import functools

import jax
import jax.numpy as jnp
from jax.experimental import pallas as pl

_DELTA_D = 1.5
_DELTA_V = 0.5
_ALPHA = 1.0
_BETA = 1.0
_GAMMA = 0.001
_K = 32


def _loss_body(emb_ref, lab_ref, out_ref):
    e = emb_ref[0]
    lab = lab_ref[0, 0]
    C, N = e.shape
    K = _K
    kiota = jax.lax.broadcasted_iota(jnp.int32, (K, N), 0)
    mask = (lab[None, :] == kiota).astype(jnp.float32)
    count = jnp.sum(mask, axis=1)
    present = count > 0.0
    n = jnp.sum(present.astype(jnp.float32))
    musum = jnp.dot(e, mask.T, preferred_element_type=jnp.float32)
    mu = musum / jnp.maximum(count, 1.0)[None, :]
    P = jnp.dot(mu.T, e, preferred_element_type=jnp.float32)
    dot_p = jnp.sum(mask * P, axis=0)
    musq = jnp.sum(mu * mu, axis=0)
    musq_p = jnp.sum(mask * musq[:, None], axis=0)
    cnt_p = jnp.sum(mask * count[:, None], axis=0)
    esq = jnp.sum(e * e, axis=0)
    d2 = esq - 2.0 * dot_p + musq_p
    norm = jnp.sqrt(jnp.maximum(d2, 1e-24))
    h = jnp.maximum(norm - _DELTA_V, 0.0) ** 2
    L_v = jnp.sum(h / cnt_p) / n

    G = jnp.dot(mu.T, mu, preferred_element_type=jnp.float32)
    pd2 = musq[:, None] + musq[None, :] - 2.0 * G
    pn = jnp.sqrt(jnp.maximum(pd2, 1e-24))
    ii = jax.lax.broadcasted_iota(jnp.int32, (K, K), 0)
    jj = jax.lax.broadcasted_iota(jnp.int32, (K, K), 1)
    margin = jnp.where(ii == jj, 0.0, 2.0 * _DELTA_D)
    pm = present[:, None] & present[None, :]
    hd = jnp.where(pm, jnp.maximum(margin - pn, 0.0) ** 2, 0.0)
    denom = jnp.maximum(n * (n - 1.0), 1.0)
    L_d = jnp.where(n > 1.0, jnp.sum(hd) / denom, 0.0)

    norms = jnp.where(present, jnp.sqrt(jnp.maximum(musq, 1e-24)), 0.0)
    L_r = jnp.sum(norms) / n

    partial = _ALPHA * L_v + _BETA * L_d + _GAMMA * L_r
    lane = jax.lax.broadcasted_iota(jnp.int32, (1, 128), 1)
    out_ref[0] = jnp.where(lane == 0, partial, 0.0)


@jax.jit
def kernel(embedded, labels):
    B, C, N = embedded.shape
    lab3 = labels.reshape(B, 1, N)
    out = pl.pallas_call(
        _loss_body,
        grid=(B,),
        in_specs=[
            pl.BlockSpec((1, C, N), lambda i: (i, 0, 0)),
            pl.BlockSpec((1, 1, N), lambda i: (i, 0, 0)),
        ],
        out_specs=pl.BlockSpec((1, 1, 128), lambda i: (i, 0, 0)),
        out_shape=jax.ShapeDtypeStruct((B, 1, 128), jnp.float32),
    )(embedded, lab3)
    return jnp.sum(out[:, 0, 0]) / B
